# baseline (speedup 1.0000x reference)
import functools

import jax
import jax.numpy as jnp
from jax import lax
from jax.experimental import pallas as pl
from jax.experimental.pallas import tpu as pltpu
from jax.experimental.pallas import tpu_sc as plsc

_MAX_SEQ_LEN = 8192
_DIM = 1024
_N = 32768

_NC = 2
_NS = 16
_NW = _NC * _NS
_B_PER_W = _N // _NW

_NSTEP = 32
_CA = 16
_CB = 32 - _CA
_NBUF = 2
_LOOK = 1


def _make_gather():
    mesh = plsc.VectorSubcoreMesh(core_axis_name="c", subcore_axis_name="s")

    @functools.partial(
        pl.kernel,
        mesh=mesh,
        out_type=jax.ShapeDtypeStruct((_N, _DIM), jnp.float32),
        scratch_types=[
            pltpu.VMEM((_NSTEP, _CA), jnp.int32),
            pltpu.VMEM((_NBUF, _CA, _DIM), jnp.float32),
            pltpu.SMEM((_NSTEP * _CB,), jnp.int32),
            pltpu.VMEM_SHARED((_NS, _NSTEP * _CB), jnp.int32),
            pltpu.VMEM_SHARED((_NBUF, _NS * _CB, _DIM), jnp.float32),
            pltpu.SemaphoreType.DMA,
            pltpu.SemaphoreType.DMA,
            pltpu.SemaphoreType.DMA,
            pltpu.SemaphoreType.DMA,
            pltpu.SemaphoreType.DMA,
            pltpu.SemaphoreType.DMA,
            pltpu.SemaphoreType.DMA,
            pltpu.SemaphoreType.DMA,
        ],
    )
    def gather(table_hbm, idxa_hbm, idxb_hbm, out_hbm,
               idxa_v, rows_v, idxb_s, idxb_sp, sp,
               ga0, ga1, oa0, oa1,
               gb0, gb1, ob0, ob1):
        gasems = (ga0, ga1)
        oasems = (oa0, oa1)
        gbsems = (gb0, gb1)
        obsems = (ob0, ob1)
        cid = lax.axis_index("c")
        sid = lax.axis_index("s")
        wid = sid * _NC + cid
        base = wid * _B_PER_W
        base_b = base + _NSTEP * _CA
        slot = sid * _CB
        pltpu.sync_copy(idxa_hbm.at[wid], idxa_v)
        pltpu.sync_copy(idxb_hbm.at[wid], idxb_sp.at[sid])
        pltpu.sync_copy(idxb_sp.at[sid], idxb_s)

        def ga_start(j, b):
            pltpu.async_copy(table_hbm.at[idxa_v.at[j]], rows_v.at[b],
                             gasems[b])

        def ga_wait(j, b):
            pltpu.make_async_copy(
                table_hbm.at[idxa_v.at[j]], rows_v.at[b], gasems[b]).wait()

        def oa_start(j, b):
            pltpu.async_copy(
                rows_v.at[b], out_hbm.at[pl.ds(base + j * _CA, _CA)],
                oasems[b])

        def oa_wait(j, b):
            pltpu.make_async_copy(
                rows_v.at[b], out_hbm.at[pl.ds(base + j * _CA, _CA)],
                oasems[b]).wait()

        def gb_start(j, b):
            def row(k, carry):
                i = idxb_s[j * _CB + k]
                pltpu.async_copy(
                    table_hbm.at[pl.ds(i, 1)],
                    sp.at[b, pl.ds(slot + k, 1)], gbsems[b])
                return carry
            lax.fori_loop(0, _CB, row, 0)

        def gb_wait(j, b):
            def row(k, carry):
                pltpu.make_async_copy(
                    table_hbm.at[pl.ds(0, 1)],
                    sp.at[b, pl.ds(slot + k, 1)], gbsems[b]).wait()
                return carry
            lax.fori_loop(0, _CB, row, 0)

        def ob_start(j, b):
            pltpu.async_copy(
                sp.at[b, pl.ds(slot, _CB)],
                out_hbm.at[pl.ds(base_b + j * _CB, _CB)], obsems[b])

        def ob_wait(j, b):
            pltpu.make_async_copy(
                sp.at[b, pl.ds(slot, _CB)],
                out_hbm.at[pl.ds(base_b + j * _CB, _CB)], obsems[b]).wait()

        paths = (
            (ga_start, ga_wait, oa_start, oa_wait),
            (gb_start, gb_wait, ob_start, ob_wait),
        )

        for s in range(_LOOK):
            for g_start, _, _, _ in paths:
                g_start(s, s % _NBUF)
        for s in range(_LOOK):
            b = s % _NBUF
            for g_start, g_wait, o_start, _ in paths:
                g_wait(s, b)
                o_start(s, b)
                g_start(s + _LOOK, (s + _LOOK) % _NBUF)

        first_steady = _LOOK
        last_steady = _NSTEP - 1 - _LOOK
        n_steady = last_steady - first_steady + 1
        n_unrolled = (n_steady // _NBUF) * _NBUF

        def body(u, carry):
            for v in range(_NBUF):
                s = first_steady + _NBUF * u + v
                b = (first_steady + v) % _NBUF
                for g_start, g_wait, o_start, o_wait in paths:
                    g_wait(s, b)
                    o_start(s, b)
                    o_wait(s - _LOOK, (first_steady + v - _LOOK) % _NBUF)
                    g_start(s + _LOOK, (first_steady + v + _LOOK) % _NBUF)
            return carry

        lax.fori_loop(0, n_unrolled // _NBUF, body, 0)
        for s in range(first_steady + n_unrolled, _NSTEP):
            b = s % _NBUF
            for g_start, g_wait, o_start, o_wait in paths:
                g_wait(s, b)
                o_start(s, b)
                o_wait(s - _LOOK, (s - _LOOK) % _NBUF)
                if s + _LOOK < _NSTEP:
                    g_start(s + _LOOK, (s + _LOOK) % _NBUF)
        for s in range(_NSTEP - _LOOK, _NSTEP):
            for _, _, _, o_wait in paths:
                o_wait(s, s % _NBUF)

    return gather


_gather = _make_gather()


def kernel(seq_len_or_indices, embedding):
    idx = seq_len_or_indices.astype(jnp.int32).reshape(_NW, _B_PER_W)
    idx_a = idx[:, : _NSTEP * _CA].reshape(_NW, _NSTEP, _CA)
    idx_b = idx[:, _NSTEP * _CA:]
    table = embedding.reshape(_MAX_SEQ_LEN, _DIM)
    return _gather(table, idx_a, idx_b)


# speedup vs baseline: 1.0183x; 1.0183x over previous
import functools

import jax
import jax.numpy as jnp
from jax import lax
from jax.experimental import pallas as pl
from jax.experimental.pallas import tpu as pltpu
from jax.experimental.pallas import tpu_sc as plsc

_MAX_SEQ_LEN = 8192
_DIM = 1024
_N = 32768

_NC = 2
_NS = 16
_NW = _NC * _NS
_B_PER_W = _N // _NW
_CHUNK = 16
_N_CHUNKS = _B_PER_W // _CHUNK
_NBUF = 6
_LOOK = 3


def _make_gather():
    mesh = plsc.VectorSubcoreMesh(core_axis_name="c", subcore_axis_name="s")

    @functools.partial(
        pl.kernel,
        mesh=mesh,
        out_type=jax.ShapeDtypeStruct((_N, _DIM), jnp.float32),
        scratch_types=[
            pltpu.SMEM((_B_PER_W,), jnp.int32),
            pltpu.VMEM_SHARED((_NS, _B_PER_W), jnp.int32),
            pltpu.VMEM_SHARED((_NBUF, _NS * _CHUNK, _DIM), jnp.float32),
            pltpu.SemaphoreType.DMA,
            pltpu.SemaphoreType.DMA,
            pltpu.SemaphoreType.DMA,
            pltpu.SemaphoreType.DMA,
            pltpu.SemaphoreType.DMA,
            pltpu.SemaphoreType.DMA,
            pltpu.SemaphoreType.DMA,
            pltpu.SemaphoreType.DMA,
            pltpu.SemaphoreType.DMA,
            pltpu.SemaphoreType.DMA,
            pltpu.SemaphoreType.DMA,
            pltpu.SemaphoreType.DMA,
        ],
    )
    def gather(table_hbm, idx_hbm, out_hbm, idx_s, idx_sp, sp,
               gsem0, gsem1, gsem2, gsem3, gsem4, gsem5,
               osem0, osem1, osem2, osem3, osem4, osem5):
        gsems = (gsem0, gsem1, gsem2, gsem3, gsem4, gsem5)
        osems = (osem0, osem1, osem2, osem3, osem4, osem5)
        cid = lax.axis_index("c")
        sid = lax.axis_index("s")
        wid = sid * _NC + cid
        base = wid * _B_PER_W
        slot = sid * _CHUNK
        pltpu.sync_copy(idx_hbm.at[wid], idx_sp.at[sid])
        pltpu.sync_copy(idx_sp.at[sid], idx_s)

        def g_start(j, b):
            def row(k, carry):
                i = idx_s[j * _CHUNK + k]
                pltpu.async_copy(
                    table_hbm.at[pl.ds(i, 1)],
                    sp.at[b, pl.ds(slot + k, 1)], gsems[b])
                return carry
            lax.fori_loop(0, _CHUNK, row, 0)

        def g_wait(j, b):
            def row(k, carry):
                pltpu.make_async_copy(
                    table_hbm.at[pl.ds(0, 1)],
                    sp.at[b, pl.ds(slot + k, 1)], gsems[b]).wait()
                return carry
            lax.fori_loop(0, _CHUNK, row, 0)

        def o_start(j, b):
            pltpu.async_copy(
                sp.at[b, pl.ds(slot, _CHUNK)],
                out_hbm.at[pl.ds(base + j * _CHUNK, _CHUNK)], osems[b])

        def o_wait(j, b):
            pltpu.make_async_copy(
                sp.at[b, pl.ds(slot, _CHUNK)],
                out_hbm.at[pl.ds(base + j * _CHUNK, _CHUNK)], osems[b]).wait()

        for s in range(_LOOK):
            g_start(s, s % _NBUF)
        for s in range(_LOOK):
            b = s % _NBUF
            g_wait(s, b)
            o_start(s, b)
            g_start(s + _LOOK, (s + _LOOK) % _NBUF)

        first_steady = _LOOK
        last_steady = _N_CHUNKS - 1 - _LOOK
        n_steady = last_steady - first_steady + 1
        n_unrolled = (n_steady // _NBUF) * _NBUF

        def body(u, carry):
            for v in range(_NBUF):
                s = first_steady + _NBUF * u + v
                b = (first_steady + v) % _NBUF
                g_wait(s, b)
                o_start(s, b)
                o_wait(s - _LOOK, (first_steady + v - _LOOK) % _NBUF)
                g_start(s + _LOOK, (first_steady + v + _LOOK) % _NBUF)
            return carry

        lax.fori_loop(0, n_unrolled // _NBUF, body, 0)
        for s in range(first_steady + n_unrolled, _N_CHUNKS):
            b = s % _NBUF
            g_wait(s, b)
            o_start(s, b)
            o_wait(s - _LOOK, (s - _LOOK) % _NBUF)
            if s + _LOOK < _N_CHUNKS:
                g_start(s + _LOOK, (s + _LOOK) % _NBUF)
        for s in range(_N_CHUNKS - _LOOK, _N_CHUNKS):
            o_wait(s, s % _NBUF)

    return gather


_gather = _make_gather()


def kernel(seq_len_or_indices, embedding):
    idx = seq_len_or_indices.astype(jnp.int32).reshape(_NW, _B_PER_W)
    table = embedding.reshape(_MAX_SEQ_LEN, _DIM)
    return _gather(table, idx)
